# baseline (speedup 1.0000x reference)
import functools
import math

import jax
import jax.numpy as jnp
from jax import lax
from jax.experimental import pallas as pl
from jax.experimental.pallas import tpu as pltpu
from jax.experimental.pallas import tpu_sc as plsc

D_MODEL = 128
SCALE = math.sqrt(float(D_MODEL))

NUM_CORES = 2
NUM_SUBCORES = 16
NW = NUM_CORES * NUM_SUBCORES
CHUNK = 128
NBUF = 4
AHEAD = 2
LANES = 16
PACKED = D_MODEL // 2


def _body(nchunks, idx_hbm, table_hbm, out_hbm, idx_v, *bufs):
    rin = bufs[:NBUF]
    rout = bufs[NBUF:2 * NBUF]
    gsem = bufs[2 * NBUF:3 * NBUF]
    ssem = bufs[3 * NBUF:]

    wid = lax.axis_index("s") * NUM_CORES + lax.axis_index("c")
    base = wid * (nchunks * CHUNK)

    pltpu.sync_copy(idx_hbm.at[wid], idx_v)

    def gather_start(j, b):
        pltpu.make_async_copy(table_hbm.at[idx_v.at[j]], rin[b], gsem[b]).start()

    def gather_wait(b):
        pltpu.make_async_copy(table_hbm.at[idx_v.at[0]], rin[b], gsem[b]).wait()

    def scatter_start(j, b):
        dst = out_hbm.at[pl.ds(base + j * CHUNK, CHUNK)]
        pltpu.make_async_copy(rout[b], dst, ssem[b]).start()

    def scatter_wait(b):
        dst = out_hbm.at[pl.ds(base, CHUNK)]
        pltpu.make_async_copy(rout[b], dst, ssem[b]).wait()

    def expand_buf(b):
        src, dst = rin[b], rout[b]

        def srow(i, carry):
            for w in range(PACKED // LANES):
                packed = src[i, pl.ds(w * LANES, LANES)]
                lo = jax.lax.bitcast_convert_type(packed << 16, jnp.float32)
                hi = jax.lax.bitcast_convert_type(
                    packed & jnp.int32(-65536), jnp.float32)
                dst[i, pl.ds(w * LANES, LANES)] = lo * SCALE
                dst[i, pl.ds(PACKED + w * LANES, LANES)] = hi * SCALE
            return carry

        lax.fori_loop(0, CHUNK, srow, 0)

    for j in range(AHEAD):
        gather_start(j, j)

    def outer(g, carry):
        for b in range(NBUF):
            j = g * NBUF + b
            gather_wait(b)
            expand_buf(b)
            scatter_start(j, b)
            jn = j + AHEAD
            bn = (b + AHEAD) % NBUF

            @pl.when(jnp.logical_and(jn < nchunks, jn >= NBUF))
            def _():
                scatter_wait(bn)

            @pl.when(jn < nchunks)
            def _():
                gather_start(jn, bn)
        return carry

    lax.fori_loop(0, nchunks // NBUF, outer, 0)

    for b in range(NBUF):
        scatter_wait(b)


def _make_call(nchunks):
    mesh = plsc.VectorSubcoreMesh(core_axis_name="c", subcore_axis_name="s")
    ntok = NW * nchunks * CHUNK
    return functools.partial(
        pl.kernel,
        mesh=mesh,
        compiler_params=pltpu.CompilerParams(use_tc_tiling_on_sc=False),
        out_type=jax.ShapeDtypeStruct((ntok, D_MODEL), jnp.float32),
        scratch_types=(
            [pltpu.VMEM((nchunks, CHUNK), jnp.int32)]
            + [pltpu.VMEM((CHUNK, PACKED), jnp.int32) for _ in range(NBUF)]
            + [pltpu.VMEM((CHUNK, D_MODEL), jnp.float32) for _ in range(NBUF)]
            + [pltpu.SemaphoreType.DMA for _ in range(2 * NBUF)]
        ),
    )(functools.partial(_body, nchunks))


def kernel(x, table):
    ntok = x.size
    assert ntok % (NW * CHUNK) == 0
    nchunks = ntok // (NW * CHUNK)
    vocab = table.shape[0]
    idx = x.reshape(NW, nchunks, CHUNK).astype(jnp.int32)
    tb = table.astype(jnp.bfloat16).reshape(vocab, 2, PACKED)
    packed = jax.lax.bitcast_convert_type(
        jnp.transpose(tb, (0, 2, 1)), jnp.int32)
    out = _make_call(nchunks)(idx, packed)
    return out.reshape(x.shape + (D_MODEL,))


# speedup vs baseline: 1.8043x; 1.8043x over previous
import functools
import math

import jax
import jax.numpy as jnp
from jax import lax
from jax.experimental import pallas as pl
from jax.experimental.pallas import tpu as pltpu
from jax.experimental.pallas import tpu_sc as plsc

D_MODEL = 128
SCALE = math.sqrt(float(D_MODEL))

NUM_CORES = 2
NUM_SUBCORES = 16
NW = NUM_CORES * NUM_SUBCORES
CHUNK = 128
NBUF = 4
AHEAD = 2
LANES = 16
PACKED = D_MODEL // 2


def _body(nchunks, idx_hbm, table_hbm, out_hbm, idx_v, *bufs):
    rin = bufs[:NBUF]
    rout = bufs[NBUF:2 * NBUF]
    gsem = bufs[2 * NBUF:3 * NBUF]
    ssem = bufs[3 * NBUF:]

    wid = lax.axis_index("s") * NUM_CORES + lax.axis_index("c")
    base = wid * (nchunks * CHUNK)

    pltpu.sync_copy(idx_hbm.at[wid], idx_v)

    def gather_start(j, b):
        pltpu.make_async_copy(table_hbm.at[idx_v.at[j]], rin[b], gsem[b]).start()

    def gather_wait(b):
        pltpu.make_async_copy(table_hbm.at[idx_v.at[0]], rin[b], gsem[b]).wait()

    def scatter_start(j, b):
        dst = out_hbm.at[pl.ds(base + j * CHUNK, CHUNK)]
        pltpu.make_async_copy(rout[b], dst, ssem[b]).start()

    def scatter_wait(b):
        dst = out_hbm.at[pl.ds(base, CHUNK)]
        pltpu.make_async_copy(rout[b], dst, ssem[b]).wait()

    def expand_buf(b):
        src, dst = rin[b], rout[b]

        @plsc.parallel_loop(0, CHUNK, unroll=4)
        def srow(i):
            for w in range(PACKED // LANES):
                packed = src[i, pl.ds(w * LANES, LANES)]
                lo = jax.lax.bitcast_convert_type(packed << 16, jnp.float32)
                hi = jax.lax.bitcast_convert_type(
                    packed & jnp.int32(-65536), jnp.float32)
                dst[i, pl.ds(w * LANES, LANES)] = lo * SCALE
                dst[i, pl.ds(PACKED + w * LANES, LANES)] = hi * SCALE

    for j in range(AHEAD):
        gather_start(j, j)

    def outer(g, carry):
        for b in range(NBUF):
            j = g * NBUF + b
            gather_wait(b)
            expand_buf(b)
            scatter_start(j, b)
            jn = j + AHEAD
            bn = (b + AHEAD) % NBUF

            @pl.when(jnp.logical_and(jn < nchunks, jn >= NBUF))
            def _():
                scatter_wait(bn)

            @pl.when(jn < nchunks)
            def _():
                gather_start(jn, bn)
        return carry

    lax.fori_loop(0, nchunks // NBUF, outer, 0)

    for b in range(NBUF):
        scatter_wait(b)


def _make_call(nchunks):
    mesh = plsc.VectorSubcoreMesh(core_axis_name="c", subcore_axis_name="s")
    ntok = NW * nchunks * CHUNK
    return functools.partial(
        pl.kernel,
        mesh=mesh,
        compiler_params=pltpu.CompilerParams(use_tc_tiling_on_sc=False),
        out_type=jax.ShapeDtypeStruct((ntok, D_MODEL), jnp.float32),
        scratch_types=(
            [pltpu.VMEM((nchunks, CHUNK), jnp.int32)]
            + [pltpu.VMEM((CHUNK, PACKED), jnp.int32) for _ in range(NBUF)]
            + [pltpu.VMEM((CHUNK, D_MODEL), jnp.float32) for _ in range(NBUF)]
            + [pltpu.SemaphoreType.DMA for _ in range(2 * NBUF)]
        ),
    )(functools.partial(_body, nchunks))


def kernel(x, table):
    ntok = x.size
    assert ntok % (NW * CHUNK) == 0
    nchunks = ntok // (NW * CHUNK)
    vocab = table.shape[0]
    idx = x.reshape(NW, nchunks, CHUNK).astype(jnp.int32)
    tb = table.astype(jnp.bfloat16).reshape(vocab, 2, PACKED)
    packed = jax.lax.bitcast_convert_type(
        jnp.transpose(tb, (0, 2, 1)), jnp.int32)
    out = _make_call(nchunks)(idx, packed)
    return out.reshape(x.shape + (D_MODEL,))


# speedup vs baseline: 1.8061x; 1.0010x over previous
import functools
import math

import jax
import jax.numpy as jnp
from jax import lax
from jax.experimental import pallas as pl
from jax.experimental.pallas import tpu as pltpu
from jax.experimental.pallas import tpu_sc as plsc

D_MODEL = 128
SCALE = math.sqrt(float(D_MODEL))

NUM_CORES = 2
NUM_SUBCORES = 16
NW = NUM_CORES * NUM_SUBCORES
CHUNK = 128
NBUF = 4
AHEAD = 2
LANES = 16
PACKED = D_MODEL // 2


def _body(nchunks, idx_hbm, table_hbm, out_hbm, idx_v, *bufs):
    rin = bufs[:NBUF]
    rout = bufs[NBUF:2 * NBUF]
    gsem = bufs[2 * NBUF:3 * NBUF]
    ssem = bufs[3 * NBUF:]

    wid = lax.axis_index("s") * NUM_CORES + lax.axis_index("c")
    base = wid * (nchunks * CHUNK)

    pltpu.sync_copy(idx_hbm.at[wid], idx_v)

    def gather_start(j, b):
        pltpu.make_async_copy(table_hbm.at[idx_v.at[j]], rin[b], gsem[b]).start()

    def gather_wait(b):
        pltpu.make_async_copy(table_hbm.at[idx_v.at[0]], rin[b], gsem[b]).wait()

    def scatter_start(j, b):
        dst = out_hbm.at[pl.ds(base + j * CHUNK, CHUNK)]
        pltpu.make_async_copy(rout[b], dst, ssem[b]).start()

    def scatter_wait(b):
        dst = out_hbm.at[pl.ds(base, CHUNK)]
        pltpu.make_async_copy(rout[b], dst, ssem[b]).wait()

    def expand_buf(b):
        src, dst = rin[b], rout[b]

        @plsc.parallel_loop(0, CHUNK, unroll=8)
        def srow(i):
            for w in range(PACKED // LANES):
                packed = src[i, pl.ds(w * LANES, LANES)]
                lo = jax.lax.bitcast_convert_type(packed << 16, jnp.float32)
                hi = jax.lax.bitcast_convert_type(packed, jnp.float32)
                dst[i, pl.ds(w * LANES, LANES)] = lo * SCALE
                dst[i, pl.ds(PACKED + w * LANES, LANES)] = hi * SCALE

    for j in range(AHEAD):
        gather_start(j, j)

    def outer(g, carry):
        for b in range(NBUF):
            j = g * NBUF + b
            gather_wait(b)
            expand_buf(b)
            scatter_start(j, b)
            jn = j + AHEAD
            bn = (b + AHEAD) % NBUF

            @pl.when(jnp.logical_and(jn < nchunks, jn >= NBUF))
            def _():
                scatter_wait(bn)

            @pl.when(jn < nchunks)
            def _():
                gather_start(jn, bn)
        return carry

    lax.fori_loop(0, nchunks // NBUF, outer, 0)

    for b in range(NBUF):
        scatter_wait(b)


def _make_call(nchunks):
    mesh = plsc.VectorSubcoreMesh(core_axis_name="c", subcore_axis_name="s")
    ntok = NW * nchunks * CHUNK
    return functools.partial(
        pl.kernel,
        mesh=mesh,
        compiler_params=pltpu.CompilerParams(use_tc_tiling_on_sc=False),
        out_type=jax.ShapeDtypeStruct((ntok, D_MODEL), jnp.float32),
        scratch_types=(
            [pltpu.VMEM((nchunks, CHUNK), jnp.int32)]
            + [pltpu.VMEM((CHUNK, PACKED), jnp.int32) for _ in range(NBUF)]
            + [pltpu.VMEM((CHUNK, D_MODEL), jnp.float32) for _ in range(NBUF)]
            + [pltpu.SemaphoreType.DMA for _ in range(2 * NBUF)]
        ),
    )(functools.partial(_body, nchunks))


def kernel(x, table):
    ntok = x.size
    assert ntok % (NW * CHUNK) == 0
    nchunks = ntok // (NW * CHUNK)
    vocab = table.shape[0]
    idx = x.reshape(NW, nchunks, CHUNK).astype(jnp.int32)
    tb = table.astype(jnp.bfloat16).reshape(vocab, 2, PACKED)
    packed = jax.lax.bitcast_convert_type(
        jnp.transpose(tb, (0, 2, 1)), jnp.int32)
    out = _make_call(nchunks)(idx, packed)
    return out.reshape(x.shape + (D_MODEL,))


# speedup vs baseline: 2.4492x; 1.3561x over previous
import functools
import math

import jax
import jax.numpy as jnp
from jax import lax
from jax.experimental import pallas as pl
from jax.experimental.pallas import tpu as pltpu
from jax.experimental.pallas import tpu_sc as plsc

D_MODEL = 128
SCALE = math.sqrt(float(D_MODEL))

NUM_CORES = 2
NUM_SUBCORES = 16
NW = NUM_CORES * NUM_SUBCORES
CHUNK = 128
NBUF = 5
AHEAD = 3
LANES = 16


def _body(nchunks, idx_hbm, table_hbm, out_hbm, idx_v, *bufs):
    rows = bufs[:NBUF]
    gsem = bufs[NBUF:2 * NBUF]
    ssem = bufs[2 * NBUF:]

    wid = lax.axis_index("s") * NUM_CORES + lax.axis_index("c")
    base = wid * (nchunks * CHUNK)

    pltpu.sync_copy(idx_hbm.at[wid], idx_v)

    def gather_start(j, b):
        pltpu.make_async_copy(table_hbm.at[idx_v.at[j]], rows[b], gsem[b]).start()

    def gather_wait(b):
        pltpu.make_async_copy(table_hbm.at[idx_v.at[0]], rows[b], gsem[b]).wait()

    def scatter_start(j, b):
        dst = out_hbm.at[pl.ds(base + j * CHUNK, CHUNK)]
        pltpu.make_async_copy(rows[b], dst, ssem[b]).start()

    def scatter_wait(b):
        dst = out_hbm.at[pl.ds(base, CHUNK)]
        pltpu.make_async_copy(rows[b], dst, ssem[b]).wait()

    def scale_buf(b):
        r = rows[b]

        def srow(i, carry):
            for c in range(D_MODEL // LANES):
                sl = (i, pl.ds(c * LANES, LANES))
                r[sl] = r[sl] * SCALE
            return carry

        lax.fori_loop(0, CHUNK, srow, 0)

    for j in range(AHEAD):
        gather_start(j, j)

    def outer(g, carry):
        for b in range(NBUF):
            j = g * NBUF + b
            gather_wait(b)
            scale_buf(b)
            scatter_start(j, b)
            jn = j + AHEAD
            bn = (b + AHEAD) % NBUF

            @pl.when(jnp.logical_and(jn < nchunks, jn >= NBUF))
            def _():
                scatter_wait(bn)

            @pl.when(jn < nchunks)
            def _():
                gather_start(jn, bn)
        return carry

    lax.fori_loop(0, nchunks // NBUF, outer, 0)

    for b in range(NBUF):
        scatter_wait(b)


def _make_call(nchunks):
    mesh = plsc.VectorSubcoreMesh(core_axis_name="c", subcore_axis_name="s")
    ntok = NW * nchunks * CHUNK
    return functools.partial(
        pl.kernel,
        mesh=mesh,
        out_type=jax.ShapeDtypeStruct((ntok, D_MODEL), jnp.float32),
        scratch_types=(
            [pltpu.VMEM((nchunks, CHUNK), jnp.int32)]
            + [pltpu.VMEM((CHUNK, D_MODEL), jnp.float32) for _ in range(NBUF)]
            + [pltpu.SemaphoreType.DMA for _ in range(2 * NBUF)]
        ),
    )(functools.partial(_body, nchunks))


def kernel(x, table):
    ntok = x.size
    assert ntok % (NW * CHUNK) == 0
    nchunks = ntok // (NW * CHUNK)
    idx = x.reshape(NW, nchunks, CHUNK).astype(jnp.int32)
    out = _make_call(nchunks)(idx, table)
    return out.reshape(x.shape + (D_MODEL,))
